# baseline (device time: 99417 ns/iter reference)
import jax
import jax.numpy as jnp
from jax import lax
from jax.experimental import pallas as pl
from jax.experimental.pallas import tpu as pltpu

N_DEV = 4
N_EXPERTS = 16
E_LOCAL = N_EXPERTS // N_DEV
N_TOK = 1024
D_IN = 512
D_OUT = 1024
CAP = 51
CHUNK = N_TOK // N_DEV
N_STEPS = 2 * (N_DEV - 1)


def kernel(x, router_W, route_idx, expert_W):
    del router_W
    my = lax.axis_index("i")

    e = route_idx[:, 0]
    onehot = (e[:, None] == jnp.arange(N_EXPERTS, dtype=e.dtype)[None, :])
    onehot = onehot.astype(jnp.float32)
    keep = (jnp.cumsum(onehot, axis=0) <= CAP).astype(jnp.float32) * onehot
    masks = lax.dynamic_slice_in_dim(keep, my * E_LOCAL, E_LOCAL, axis=1)

    def body(x_ref, w_ref, m_ref, out_ref, comm_ref, send_sems, recv_sems):
        r = lax.axis_index("i")
        left = lax.rem(r + N_DEV - 1, N_DEV)
        right = lax.rem(r + 1, N_DEV)

        barrier_sem = pltpu.get_barrier_semaphore()
        for nbr in (left, right):
            pl.semaphore_signal(
                barrier_sem, inc=1,
                device_id=(nbr,), device_id_type=pl.DeviceIdType.MESH,
            )
        pl.semaphore_wait(barrier_sem, 2)

        acc = jnp.dot(
            x_ref[:, :] * m_ref[:, 0:1], w_ref[0],
            preferred_element_type=jnp.float32,
        )
        for j in range(1, E_LOCAL):
            acc = acc + jnp.dot(
                x_ref[:, :] * m_ref[:, j : j + 1], w_ref[j],
                preferred_element_type=jnp.float32,
            )
        out_ref[:, :] = acc

        for s in range(N_STEPS):
            if s < N_DEV - 1:
                send_c = lax.rem(r + 2 * N_DEV - s, N_DEV)
                recv_c = lax.rem(r + 2 * N_DEV - s - 1, N_DEV)
            else:
                t = s - (N_DEV - 1)
                send_c = lax.rem(r + 1 + 2 * N_DEV - t, N_DEV)
                recv_c = lax.rem(r + 2 * N_DEV - t, N_DEV)

            rdma = pltpu.make_async_remote_copy(
                src_ref=out_ref.at[pl.ds(send_c * CHUNK, CHUNK), :],
                dst_ref=comm_ref.at[s],
                send_sem=send_sems.at[s],
                recv_sem=recv_sems.at[s],
                device_id=(right,),
                device_id_type=pl.DeviceIdType.MESH,
            )
            rdma.start()
            rdma.wait()

            if s < N_DEV - 1:
                out_ref[pl.ds(recv_c * CHUNK, CHUNK), :] = (
                    out_ref[pl.ds(recv_c * CHUNK, CHUNK), :] + comm_ref[s]
                )
            else:
                out_ref[pl.ds(recv_c * CHUNK, CHUNK), :] = comm_ref[s]

    return pl.pallas_call(
        body,
        out_shape=jax.ShapeDtypeStruct((N_TOK, D_OUT), jnp.float32),
        in_specs=[
            pl.BlockSpec(memory_space=pltpu.VMEM),
            pl.BlockSpec(memory_space=pltpu.VMEM),
            pl.BlockSpec(memory_space=pltpu.VMEM),
        ],
        out_specs=pl.BlockSpec(memory_space=pltpu.VMEM),
        scratch_shapes=[
            pltpu.VMEM((N_STEPS, CHUNK, D_OUT), jnp.float32),
            pltpu.SemaphoreType.DMA((N_STEPS,)),
            pltpu.SemaphoreType.DMA((N_STEPS,)),
        ],
        compiler_params=pltpu.CompilerParams(collective_id=0),
    )(x, expert_W, masks)


# device time: 58718 ns/iter; 1.6931x vs baseline; 1.6931x over previous
import jax
import jax.numpy as jnp
from jax import lax
from jax.experimental import pallas as pl
from jax.experimental.pallas import tpu as pltpu

N_DEV = 4
N_EXPERTS = 16
E_LOCAL = N_EXPERTS // N_DEV
N_TOK = 1024
D_IN = 512
D_OUT = 1024
CAP = 51
SLOT = 64
BLOCK = E_LOCAL * SLOT
N_HOPS = N_DEV - 1


def kernel(x, router_W, route_idx, expert_W):
    del router_W
    my = lax.axis_index("i")

    e = route_idx[:, 0]
    onehot = (e[:, None] == jnp.arange(N_EXPERTS, dtype=e.dtype)[None, :])
    rank = jnp.sum(jnp.cumsum(onehot.astype(jnp.float32), axis=0) * onehot,
                   axis=1).astype(jnp.int32) - 1
    kept = rank < CAP
    gslot = SLOT * e + rank

    slots = jnp.arange(N_DEV * BLOCK, dtype=jnp.int32)
    P = ((gslot[:, None] == slots[None, :]) & kept[:, None]).astype(jnp.float32)
    G = lax.dynamic_slice_in_dim(P, my * BLOCK, BLOCK, axis=1).T

    def body(x_ref, w_ref, g_ref, p_ref, out_ref, comm_ref,
             send_sems, recv_sems):
        r = lax.axis_index("i")
        left = lax.rem(r + N_DEV - 1, N_DEV)
        right = lax.rem(r + 1, N_DEV)

        barrier_sem = pltpu.get_barrier_semaphore()
        for nbr in (left, right):
            pl.semaphore_signal(
                barrier_sem, inc=1,
                device_id=(nbr,), device_id_type=pl.DeviceIdType.MESH,
            )
        pl.semaphore_wait(barrier_sem, 2)

        cx = jnp.dot(g_ref[:, :], x_ref[:, :],
                     preferred_element_type=jnp.float32)
        block = jnp.concatenate(
            [
                jnp.dot(cx[j * SLOT:(j + 1) * SLOT, :], w_ref[j],
                        preferred_element_type=jnp.float32)
                for j in range(E_LOCAL)
            ],
            axis=0,
        )
        comm_ref[0] = block

        out_ref[:, :] = jnp.dot(
            p_ref[:, pl.ds(r * BLOCK, BLOCK)], block,
            preferred_element_type=jnp.float32,
        )

        for h in range(N_HOPS):
            rdma = pltpu.make_async_remote_copy(
                src_ref=comm_ref.at[h],
                dst_ref=comm_ref.at[h + 1],
                send_sem=send_sems.at[h],
                recv_sem=recv_sems.at[h],
                device_id=(right,),
                device_id_type=pl.DeviceIdType.MESH,
            )
            rdma.start()
            rdma.wait()

            q = lax.rem(r + 2 * N_DEV - 1 - h, N_DEV)
            out_ref[:, :] = out_ref[:, :] + jnp.dot(
                p_ref[:, pl.ds(q * BLOCK, BLOCK)], comm_ref[h + 1],
                preferred_element_type=jnp.float32,
            )

    return pl.pallas_call(
        body,
        out_shape=jax.ShapeDtypeStruct((N_TOK, D_OUT), jnp.float32),
        in_specs=[
            pl.BlockSpec(memory_space=pltpu.VMEM),
            pl.BlockSpec(memory_space=pltpu.VMEM),
            pl.BlockSpec(memory_space=pltpu.VMEM),
            pl.BlockSpec(memory_space=pltpu.VMEM),
        ],
        out_specs=pl.BlockSpec(memory_space=pltpu.VMEM),
        scratch_shapes=[
            pltpu.VMEM((N_DEV, BLOCK, D_OUT), jnp.float32),
            pltpu.SemaphoreType.DMA((N_HOPS,)),
            pltpu.SemaphoreType.DMA((N_HOPS,)),
        ],
        compiler_params=pltpu.CompilerParams(collective_id=0),
    )(x, expert_W, G, P)


# device time: 53151 ns/iter; 1.8705x vs baseline; 1.1047x over previous
import jax
import jax.numpy as jnp
from jax import lax
from jax.experimental import pallas as pl
from jax.experimental.pallas import tpu as pltpu

N_DEV = 4
N_EXPERTS = 16
E_LOCAL = N_EXPERTS // N_DEV
N_TOK = 1024
D_IN = 512
D_OUT = 1024
CAP = 51
SLOT = 56
BLOCK = E_LOCAL * SLOT
N_HOPS = N_DEV - 1


def kernel(x, router_W, route_idx, expert_W):
    del router_W
    my = lax.axis_index("i")

    e = route_idx[:, 0]
    onehot = (e[:, None] == jnp.arange(N_EXPERTS, dtype=e.dtype)[None, :])
    rank = jnp.sum(jnp.cumsum(onehot.astype(jnp.float32), axis=0) * onehot,
                   axis=1).astype(jnp.int32) - 1
    kept = rank < CAP
    gslot = SLOT * e + rank

    slots = jnp.arange(N_DEV * BLOCK, dtype=jnp.int32)
    P = ((gslot[:, None] == slots[None, :]) & kept[:, None]).astype(jnp.float32)
    P = P.reshape(N_TOK, N_DEV, BLOCK).transpose(1, 0, 2)
    G = lax.dynamic_index_in_dim(P, my, axis=0, keepdims=False).T

    def body(x_ref, w_ref, g_ref, p_ref, out_ref, comm_ref,
             send_sems, recv_sems):
        r = lax.axis_index("i")
        left = lax.rem(r + N_DEV - 1, N_DEV)
        right = lax.rem(r + 1, N_DEV)

        barrier_sem = pltpu.get_barrier_semaphore()
        for nbr in (left, right):
            pl.semaphore_signal(
                barrier_sem, inc=1,
                device_id=(nbr,), device_id_type=pl.DeviceIdType.MESH,
            )
        pl.semaphore_wait(barrier_sem, 2)

        cx = jnp.dot(g_ref[:, :], x_ref[:, :],
                     preferred_element_type=jnp.float32)
        block = jnp.concatenate(
            [
                jnp.dot(cx[j * SLOT:(j + 1) * SLOT, :], w_ref[j],
                        preferred_element_type=jnp.float32)
                for j in range(E_LOCAL)
            ],
            axis=0,
        )
        comm_ref[0] = block

        def hop(h):
            rdma = pltpu.make_async_remote_copy(
                src_ref=comm_ref.at[h],
                dst_ref=comm_ref.at[h + 1],
                send_sem=send_sems.at[h],
                recv_sem=recv_sems.at[h],
                device_id=(right,),
                device_id_type=pl.DeviceIdType.MESH,
            )
            rdma.start()
            return rdma

        rdmas = [hop(0)]
        out_ref[:, :] = jnp.dot(p_ref[r], block,
                                preferred_element_type=jnp.float32)

        for h in range(N_HOPS):
            rdmas[h].wait_recv()
            if h + 1 < N_HOPS:
                rdmas.append(hop(h + 1))
            q = lax.rem(r + 2 * N_DEV - 1 - h, N_DEV)
            out_ref[:, :] = out_ref[:, :] + jnp.dot(
                p_ref[q], comm_ref[h + 1],
                preferred_element_type=jnp.float32,
            )

        for rdma in rdmas:
            rdma.wait_send()

    return pl.pallas_call(
        body,
        out_shape=jax.ShapeDtypeStruct((N_TOK, D_OUT), jnp.float32),
        in_specs=[
            pl.BlockSpec(memory_space=pltpu.VMEM),
            pl.BlockSpec(memory_space=pltpu.VMEM),
            pl.BlockSpec(memory_space=pltpu.VMEM),
            pl.BlockSpec(memory_space=pltpu.VMEM),
        ],
        out_specs=pl.BlockSpec(memory_space=pltpu.VMEM),
        scratch_shapes=[
            pltpu.VMEM((N_DEV, BLOCK, D_OUT), jnp.float32),
            pltpu.SemaphoreType.DMA((N_HOPS,)),
            pltpu.SemaphoreType.DMA((N_HOPS,)),
        ],
        compiler_params=pltpu.CompilerParams(collective_id=0),
    )(x, expert_W, G, P)


# device time: 51649 ns/iter; 1.9249x vs baseline; 1.0291x over previous
import jax
import jax.numpy as jnp
from jax import lax
from jax.experimental import pallas as pl
from jax.experimental.pallas import tpu as pltpu

N_DEV = 4
N_EXPERTS = 16
E_LOCAL = N_EXPERTS // N_DEV
N_TOK = 1024
D_IN = 512
D_OUT = 1024
CAP = 51
SLOT = 56
BLOCK = E_LOCAL * SLOT
N_HOPS = N_DEV - 1


def kernel(x, router_W, route_idx, expert_W):
    del router_W

    e = route_idx[:, 0]
    onehot = (e[:, None] == jnp.arange(N_EXPERTS, dtype=e.dtype)[None, :])
    rank = jnp.sum(jnp.cumsum(onehot.astype(jnp.float32), axis=0) * onehot,
                   axis=1).astype(jnp.int32) - 1
    gslot = jnp.where(rank < CAP, SLOT * e + rank, -1)
    gslot_col = gslot.reshape(N_TOK, 1)
    gslot_row = gslot.reshape(1, N_TOK)

    def body(x_ref, w_ref, sc_ref, sr_ref, out_ref, comm_ref,
             send_sems, recv_sems):
        r = lax.axis_index("i")
        left = lax.rem(r + N_DEV - 1, N_DEV)
        right = lax.rem(r + 1, N_DEV)

        barrier_sem = pltpu.get_barrier_semaphore()
        for nbr in (left, right):
            pl.semaphore_signal(
                barrier_sem, inc=1,
                device_id=(nbr,), device_id_type=pl.DeviceIdType.MESH,
            )

        slot_iota_r = lax.broadcasted_iota(jnp.int32, (BLOCK, N_TOK), 0)
        g = (slot_iota_r + r * BLOCK == sr_ref[:, :]).astype(jnp.float32)

        cx = jnp.dot(g, x_ref[:, :],
                     preferred_element_type=jnp.float32)
        block = jnp.concatenate(
            [
                jnp.dot(cx[j * SLOT:(j + 1) * SLOT, :], w_ref[j],
                        preferred_element_type=jnp.float32)
                for j in range(E_LOCAL)
            ],
            axis=0,
        )
        comm_ref[0] = block

        pl.semaphore_wait(barrier_sem, 2)

        def hop(h):
            rdma = pltpu.make_async_remote_copy(
                src_ref=comm_ref.at[h],
                dst_ref=comm_ref.at[h + 1],
                send_sem=send_sems.at[h],
                recv_sem=recv_sems.at[h],
                device_id=(right,),
                device_id_type=pl.DeviceIdType.MESH,
            )
            rdma.start()
            return rdma

        slot_iota = lax.broadcasted_iota(jnp.int32, (N_TOK, BLOCK), 1)

        def scatter_mat(q):
            return (slot_iota + q * BLOCK == sc_ref[:, :]).astype(jnp.float32)

        rdmas = [hop(0)]
        out_ref[:, :] = jnp.dot(scatter_mat(r), block,
                                preferred_element_type=jnp.float32)

        for h in range(N_HOPS):
            rdmas[h].wait_recv()
            if h + 1 < N_HOPS:
                rdmas.append(hop(h + 1))
            q = lax.rem(r + 2 * N_DEV - 1 - h, N_DEV)
            out_ref[:, :] = out_ref[:, :] + jnp.dot(
                scatter_mat(q), comm_ref[h + 1],
                preferred_element_type=jnp.float32,
            )

        for rdma in rdmas:
            rdma.wait_send()

    return pl.pallas_call(
        body,
        out_shape=jax.ShapeDtypeStruct((N_TOK, D_OUT), jnp.float32),
        in_specs=[
            pl.BlockSpec(memory_space=pltpu.VMEM),
            pl.BlockSpec(memory_space=pltpu.VMEM),
            pl.BlockSpec(memory_space=pltpu.VMEM),
            pl.BlockSpec(memory_space=pltpu.VMEM),
        ],
        out_specs=pl.BlockSpec(memory_space=pltpu.VMEM),
        scratch_shapes=[
            pltpu.VMEM((N_DEV, BLOCK, D_OUT), jnp.float32),
            pltpu.SemaphoreType.DMA((N_HOPS,)),
            pltpu.SemaphoreType.DMA((N_HOPS,)),
        ],
        compiler_params=pltpu.CompilerParams(collective_id=0),
    )(x, expert_W, gslot_col, gslot_row)


# device time: 48020 ns/iter; 2.0703x vs baseline; 1.0756x over previous
import jax
import jax.numpy as jnp
from jax import lax
from jax.experimental import pallas as pl
from jax.experimental.pallas import tpu as pltpu

N_DEV = 4
N_EXPERTS = 16
E_LOCAL = N_EXPERTS // N_DEV
N_TOK = 1024
D_IN = 512
D_OUT = 1024
CAP = 51
SLOT = 56
BLOCK = E_LOCAL * SLOT
N_HOPS = N_DEV - 1


def kernel(x, router_W, route_idx, expert_W):
    del router_W
    route_idx_row = route_idx.reshape(1, N_TOK)

    def body(x_ref, w_ref, sc_ref, sr_ref, out_ref, comm_ref,
             send_sems, recv_sems):
        r = lax.axis_index("i")
        left = lax.rem(r + N_DEV - 1, N_DEV)
        right = lax.rem(r + 1, N_DEV)

        barrier_sem = pltpu.get_barrier_semaphore()
        for nbr in (left, right):
            pl.semaphore_signal(
                barrier_sem, inc=1,
                device_id=(nbr,), device_id_type=pl.DeviceIdType.MESH,
            )

        idx_c = sc_ref[:, :]
        idx_r = sr_ref[:, :]
        row_i = lax.broadcasted_iota(jnp.int32, (N_TOK, N_TOK), 0)
        col_i = lax.broadcasted_iota(jnp.int32, (N_TOK, N_TOK), 1)
        tri_lo = (col_i <= row_i).astype(jnp.float32)
        tri_up = (row_i <= col_i).astype(jnp.float32)

        oh_c = (idx_c == lax.broadcasted_iota(
            jnp.int32, (N_TOK, N_EXPERTS), 1)).astype(jnp.float32)
        cum_c = jnp.dot(tri_lo, oh_c, preferred_element_type=jnp.float32)
        rank_c = jnp.sum(cum_c * oh_c, axis=1, keepdims=True
                         ).astype(jnp.int32) - 1
        gslot_c = jnp.where(rank_c < CAP, SLOT * idx_c + rank_c, -1)

        oh_r = (idx_r == lax.broadcasted_iota(
            jnp.int32, (N_EXPERTS, N_TOK), 0)).astype(jnp.float32)
        cum_r = jnp.dot(oh_r, tri_up, preferred_element_type=jnp.float32)
        rank_r = jnp.sum(cum_r * oh_r, axis=0, keepdims=True
                         ).astype(jnp.int32) - 1
        gslot_r = jnp.where(rank_r < CAP, SLOT * idx_r + rank_r, -1)

        slot_iota_r = lax.broadcasted_iota(jnp.int32, (BLOCK, N_TOK), 0)
        g = (slot_iota_r + r * BLOCK == gslot_r).astype(jnp.float32)

        cx = jnp.dot(g, x_ref[:, :],
                     preferred_element_type=jnp.float32)
        block = jnp.concatenate(
            [
                jnp.dot(cx[j * SLOT:(j + 1) * SLOT, :], w_ref[j],
                        preferred_element_type=jnp.float32)
                for j in range(E_LOCAL)
            ],
            axis=0,
        )
        comm_ref[0] = block

        pl.semaphore_wait(barrier_sem, 2)

        def hop(h):
            rdma = pltpu.make_async_remote_copy(
                src_ref=comm_ref.at[h],
                dst_ref=comm_ref.at[h + 1],
                send_sem=send_sems.at[h],
                recv_sem=recv_sems.at[h],
                device_id=(right,),
                device_id_type=pl.DeviceIdType.MESH,
            )
            rdma.start()
            return rdma

        slot_iota = lax.broadcasted_iota(jnp.int32, (N_TOK, BLOCK), 1)

        def scatter_mat(q):
            return (slot_iota + q * BLOCK == gslot_c).astype(jnp.float32)

        rdmas = [hop(0)]
        out_ref[:, :] = jnp.dot(scatter_mat(r), block,
                                preferred_element_type=jnp.float32)

        for h in range(N_HOPS):
            rdmas[h].wait_recv()
            if h + 1 < N_HOPS:
                rdmas.append(hop(h + 1))
            q = lax.rem(r + 2 * N_DEV - 1 - h, N_DEV)
            out_ref[:, :] = out_ref[:, :] + jnp.dot(
                scatter_mat(q), comm_ref[h + 1],
                preferred_element_type=jnp.float32,
            )

        for rdma in rdmas:
            rdma.wait_send()

    return pl.pallas_call(
        body,
        out_shape=jax.ShapeDtypeStruct((N_TOK, D_OUT), jnp.float32),
        in_specs=[
            pl.BlockSpec(memory_space=pltpu.VMEM),
            pl.BlockSpec(memory_space=pltpu.VMEM),
            pl.BlockSpec(memory_space=pltpu.VMEM),
            pl.BlockSpec(memory_space=pltpu.VMEM),
        ],
        out_specs=pl.BlockSpec(memory_space=pltpu.VMEM),
        scratch_shapes=[
            pltpu.VMEM((N_DEV, BLOCK, D_OUT), jnp.float32),
            pltpu.SemaphoreType.DMA((N_HOPS,)),
            pltpu.SemaphoreType.DMA((N_HOPS,)),
        ],
        compiler_params=pltpu.CompilerParams(collective_id=0),
    )(x, expert_W, route_idx, route_idx_row)


# device time: 44868 ns/iter; 2.2158x vs baseline; 1.0703x over previous
import jax
import jax.numpy as jnp
from jax import lax
from jax.experimental import pallas as pl
from jax.experimental.pallas import tpu as pltpu

N_DEV = 4
N_EXPERTS = 16
E_LOCAL = N_EXPERTS // N_DEV
N_TOK = 1024
D_IN = 512
D_OUT = 1024
CAP = 51
SLOT = 56
BLOCK = E_LOCAL * SLOT
HALF = BLOCK // 2
N_HOPS = N_DEV - 1


def kernel(x, router_W, route_idx, expert_W):
    del router_W
    route_idx_row = route_idx.reshape(1, N_TOK)

    def body(x_ref, w_ref, sc_ref, sr_ref, out_ref, comm_ref,
             send_sems, recv_sems):
        r = lax.axis_index("i")
        left = lax.rem(r + N_DEV - 1, N_DEV)
        right = lax.rem(r + 1, N_DEV)

        barrier_sem = pltpu.get_barrier_semaphore()
        for nbr in (left, right):
            pl.semaphore_signal(
                barrier_sem, inc=1,
                device_id=(nbr,), device_id_type=pl.DeviceIdType.MESH,
            )

        idx_c = sc_ref[:, :]
        idx_r = sr_ref[:, :]
        row_i = lax.broadcasted_iota(jnp.int32, (N_TOK, N_TOK), 0)
        col_i = lax.broadcasted_iota(jnp.int32, (N_TOK, N_TOK), 1)
        tri_lo = (col_i <= row_i).astype(jnp.float32)
        tri_up = (row_i <= col_i).astype(jnp.float32)

        oh_c = (idx_c == lax.broadcasted_iota(
            jnp.int32, (N_TOK, N_EXPERTS), 1)).astype(jnp.float32)
        cum_c = jnp.dot(tri_lo, oh_c, preferred_element_type=jnp.float32)
        rank_c = jnp.sum(cum_c * oh_c, axis=1, keepdims=True
                         ).astype(jnp.int32) - 1
        gslot_c = jnp.where(rank_c < CAP, SLOT * idx_c + rank_c, -1)

        oh_r = (idx_r == lax.broadcasted_iota(
            jnp.int32, (N_EXPERTS, N_TOK), 0)).astype(jnp.float32)
        cum_r = jnp.dot(oh_r, tri_up, preferred_element_type=jnp.float32)
        rank_r = jnp.sum(cum_r * oh_r, axis=0, keepdims=True
                         ).astype(jnp.int32) - 1
        gslot_r = jnp.where(rank_r < CAP, SLOT * idx_r + rank_r, -1)

        slot_iota_r = lax.broadcasted_iota(jnp.int32, (BLOCK, N_TOK), 0)
        g = (slot_iota_r + r * BLOCK == gslot_r).astype(jnp.float32)

        cx = jnp.dot(g, x_ref[:, :],
                     preferred_element_type=jnp.float32)
        block = jnp.concatenate(
            [
                jnp.dot(cx[j * SLOT:(j + 1) * SLOT, :], w_ref[j],
                        preferred_element_type=jnp.float32)
                for j in range(E_LOCAL)
            ],
            axis=0,
        )
        comm_ref[0] = block

        pl.semaphore_wait(barrier_sem, 2)

        def hop(h, c):
            rows = pl.ds(c * HALF, HALF)
            rdma = pltpu.make_async_remote_copy(
                src_ref=comm_ref.at[h, rows],
                dst_ref=comm_ref.at[h + 1, rows],
                send_sem=send_sems.at[h, c],
                recv_sem=recv_sems.at[h, c],
                device_id=(right,),
                device_id_type=pl.DeviceIdType.MESH,
            )
            rdma.start()
            return rdma

        slot_iota = lax.broadcasted_iota(jnp.int32, (N_TOK, BLOCK), 1)

        def scatter_mat(q):
            return (slot_iota + q * BLOCK == gslot_c).astype(jnp.float32)

        rdmas = {(0, 0): hop(0, 0), (0, 1): hop(0, 1)}
        out_ref[:, :] = jnp.dot(scatter_mat(r), block,
                                preferred_element_type=jnp.float32)

        for h in range(N_HOPS):
            q = lax.rem(r + 2 * N_DEV - 1 - h, N_DEV)
            pq = scatter_mat(q)
            for c in range(2):
                rdmas[(h, c)].wait_recv()
                if h + 1 < N_HOPS:
                    rdmas[(h + 1, c)] = hop(h + 1, c)
                out_ref[:, :] = out_ref[:, :] + jnp.dot(
                    pq[:, c * HALF:(c + 1) * HALF],
                    comm_ref[h + 1, c * HALF:(c + 1) * HALF, :],
                    preferred_element_type=jnp.float32,
                )

        for rdma in rdmas.values():
            rdma.wait_send()

    return pl.pallas_call(
        body,
        out_shape=jax.ShapeDtypeStruct((N_TOK, D_OUT), jnp.float32),
        in_specs=[
            pl.BlockSpec(memory_space=pltpu.VMEM),
            pl.BlockSpec(memory_space=pltpu.VMEM),
            pl.BlockSpec(memory_space=pltpu.VMEM),
            pl.BlockSpec(memory_space=pltpu.VMEM),
        ],
        out_specs=pl.BlockSpec(memory_space=pltpu.VMEM),
        scratch_shapes=[
            pltpu.VMEM((N_DEV, BLOCK, D_OUT), jnp.float32),
            pltpu.SemaphoreType.DMA((N_HOPS, 2)),
            pltpu.SemaphoreType.DMA((N_HOPS, 2)),
        ],
        compiler_params=pltpu.CompilerParams(collective_id=0),
    )(x, expert_W, route_idx, route_idx_row)


# device time: 29781 ns/iter; 3.3383x vs baseline; 1.5066x over previous
import jax
import jax.numpy as jnp
from jax import lax
from jax.experimental import pallas as pl
from jax.experimental.pallas import tpu as pltpu

N_DEV = 4
N_EXPERTS = 16
E_LOCAL = N_EXPERTS // N_DEV
N_TOK = 1024
D_IN = 512
D_OUT = 1024
CAP = 51
SLOT = 56
BLOCK = E_LOCAL * SLOT
HALF = BLOCK // 2
N_HOPS = N_DEV - 1


def kernel(x, router_W, route_idx, expert_W):
    del router_W
    route_idx_row = route_idx.reshape(1, N_TOK)

    def body(x_ref, w_ref, sc_ref, sr_ref, out_ref, comm_ref,
             send_sems, recv_sems):
        r = lax.axis_index("i")
        left = lax.rem(r + N_DEV - 1, N_DEV)
        right = lax.rem(r + 1, N_DEV)

        barrier_sem = pltpu.get_barrier_semaphore()
        for nbr in (left, right):
            pl.semaphore_signal(
                barrier_sem, inc=1,
                device_id=(nbr,), device_id_type=pl.DeviceIdType.MESH,
            )

        idx_c = sc_ref[:, :]
        idx_r = sr_ref[:, :]
        row_i = lax.broadcasted_iota(jnp.int32, (N_TOK, N_TOK), 0)
        col_i = lax.broadcasted_iota(jnp.int32, (N_TOK, N_TOK), 1)
        tri_lo = (col_i <= row_i).astype(jnp.float32)
        tri_up = (row_i <= col_i).astype(jnp.float32)

        oh_c = (idx_c == lax.broadcasted_iota(
            jnp.int32, (N_TOK, N_EXPERTS), 1)).astype(jnp.float32)
        cum_c = jnp.dot(tri_lo, oh_c, preferred_element_type=jnp.float32)
        rank_c = jnp.sum(cum_c * oh_c, axis=1, keepdims=True
                         ).astype(jnp.int32) - 1
        gslot_c = jnp.where(rank_c < CAP, SLOT * idx_c + rank_c, -1)

        oh_r = (idx_r == lax.broadcasted_iota(
            jnp.int32, (N_EXPERTS, N_TOK), 0)).astype(jnp.float32)
        cum_r = jnp.dot(oh_r, tri_up, preferred_element_type=jnp.float32)
        rank_r = jnp.sum(cum_r * oh_r, axis=0, keepdims=True
                         ).astype(jnp.int32) - 1
        gslot_r = jnp.where(rank_r < CAP, SLOT * idx_r + rank_r, -1)

        slot_iota_r = lax.broadcasted_iota(jnp.int32, (BLOCK, N_TOK), 0)
        g = (slot_iota_r + r * BLOCK == gslot_r).astype(jnp.float32)

        cx = jnp.dot(g, x_ref[:, :],
                     preferred_element_type=jnp.float32)
        block = jnp.concatenate(
            [
                jnp.dot(cx[j * SLOT:(j + 1) * SLOT, :], w_ref[j],
                        preferred_element_type=jnp.float32)
                for j in range(E_LOCAL)
            ],
            axis=0,
        )
        comm_ref[0] = block.astype(jnp.bfloat16)

        pl.semaphore_wait(barrier_sem, 2)

        def hop(h, c):
            rows = pl.ds(c * HALF, HALF)
            rdma = pltpu.make_async_remote_copy(
                src_ref=comm_ref.at[h, rows],
                dst_ref=comm_ref.at[h + 1, rows],
                send_sem=send_sems.at[h, c],
                recv_sem=recv_sems.at[h, c],
                device_id=(right,),
                device_id_type=pl.DeviceIdType.MESH,
            )
            rdma.start()
            return rdma

        slot_iota = lax.broadcasted_iota(jnp.int32, (N_TOK, BLOCK), 1)

        def scatter_mat(q, dtype=jnp.float32):
            return (slot_iota + q * BLOCK == gslot_c).astype(dtype)

        rdmas = {(0, 0): hop(0, 0), (0, 1): hop(0, 1)}
        out_ref[:, :] = jnp.dot(scatter_mat(r), block,
                                preferred_element_type=jnp.float32)

        for h in range(N_HOPS):
            q = lax.rem(r + 2 * N_DEV - 1 - h, N_DEV)
            pq = scatter_mat(q, jnp.bfloat16)
            for c in range(2):
                rdmas[(h, c)].wait_recv()
                if h + 1 < N_HOPS:
                    rdmas[(h + 1, c)] = hop(h + 1, c)
                out_ref[:, :] = out_ref[:, :] + jnp.dot(
                    pq[:, c * HALF:(c + 1) * HALF],
                    comm_ref[h + 1, c * HALF:(c + 1) * HALF, :],
                    preferred_element_type=jnp.float32,
                )

        for rdma in rdmas.values():
            rdma.wait_send()

    return pl.pallas_call(
        body,
        out_shape=jax.ShapeDtypeStruct((N_TOK, D_OUT), jnp.float32),
        in_specs=[
            pl.BlockSpec(memory_space=pltpu.VMEM),
            pl.BlockSpec(memory_space=pltpu.VMEM),
            pl.BlockSpec(memory_space=pltpu.VMEM),
            pl.BlockSpec(memory_space=pltpu.VMEM),
        ],
        out_specs=pl.BlockSpec(memory_space=pltpu.VMEM),
        scratch_shapes=[
            pltpu.VMEM((N_DEV, BLOCK, D_OUT), jnp.bfloat16),
            pltpu.SemaphoreType.DMA((N_HOPS, 2)),
            pltpu.SemaphoreType.DMA((N_HOPS, 2)),
        ],
        compiler_params=pltpu.CompilerParams(collective_id=0),
    )(x, expert_W, route_idx, route_idx_row)


# device time: 29203 ns/iter; 3.4043x vs baseline; 1.0198x over previous
import jax
import jax.numpy as jnp
from jax import lax
from jax.experimental import pallas as pl
from jax.experimental.pallas import tpu as pltpu

N_DEV = 4
N_EXPERTS = 16
E_LOCAL = N_EXPERTS // N_DEV
N_TOK = 1024
D_IN = 512
D_OUT = 1024
CAP = 51
SLOT = 56
BLOCK = E_LOCAL * SLOT
HALF = BLOCK // 2
N_HOPS = N_DEV - 1


def kernel(x, router_W, route_idx, expert_W):
    del router_W
    route_idx_row = route_idx.reshape(1, N_TOK)

    def body(x_ref, w_ref, sc_ref, sr_ref, out_ref, comm_ref,
             send_sems, recv_sems):
        r = lax.axis_index("i")
        left = lax.rem(r + N_DEV - 1, N_DEV)
        right = lax.rem(r + 1, N_DEV)

        barrier_sem = pltpu.get_barrier_semaphore()
        for nbr in (left, right):
            pl.semaphore_signal(
                barrier_sem, inc=1,
                device_id=(nbr,), device_id_type=pl.DeviceIdType.MESH,
            )

        idx_c = sc_ref[:, :]
        idx_r = sr_ref[:, :]
        row_i = lax.broadcasted_iota(jnp.int32, (N_TOK, N_TOK), 0)
        col_i = lax.broadcasted_iota(jnp.int32, (N_TOK, N_TOK), 1)

        oh_r = (idx_r == lax.broadcasted_iota(
            jnp.int32, (N_EXPERTS, N_TOK), 0)).astype(jnp.float32)
        tri_up = (row_i <= col_i).astype(jnp.float32)
        cum_r = jnp.dot(oh_r, tri_up, preferred_element_type=jnp.float32)
        rank_r = jnp.sum(cum_r * oh_r, axis=0, keepdims=True
                         ).astype(jnp.int32) - 1
        gslot_r = jnp.where(rank_r < CAP, SLOT * idx_r + rank_r, -1)

        slot_iota_r = lax.broadcasted_iota(jnp.int32, (BLOCK, N_TOK), 0)
        g = (slot_iota_r + r * BLOCK == gslot_r).astype(jnp.float32)

        cx = jnp.dot(g, x_ref[:, :],
                     preferred_element_type=jnp.float32)

        def expert_rows(j):
            return jnp.dot(cx[j * SLOT:(j + 1) * SLOT, :], w_ref[j],
                           preferred_element_type=jnp.float32)

        def hop(h, c):
            rows = pl.ds(c * HALF, HALF)
            rdma = pltpu.make_async_remote_copy(
                src_ref=comm_ref.at[h, rows],
                dst_ref=comm_ref.at[h + 1, rows],
                send_sem=send_sems.at[h, c],
                recv_sem=recv_sems.at[h, c],
                device_id=(right,),
                device_id_type=pl.DeviceIdType.MESH,
            )
            rdma.start()
            return rdma

        half0 = jnp.concatenate([expert_rows(0), expert_rows(1)], axis=0)
        comm_ref[0, 0:HALF, :] = half0.astype(jnp.bfloat16)
        pl.semaphore_wait(barrier_sem, 2)
        rdmas = {(0, 0): hop(0, 0)}

        half1 = jnp.concatenate([expert_rows(2), expert_rows(3)], axis=0)
        comm_ref[0, HALF:BLOCK, :] = half1.astype(jnp.bfloat16)
        rdmas[(0, 1)] = hop(0, 1)

        oh_c = (idx_c == lax.broadcasted_iota(
            jnp.int32, (N_TOK, N_EXPERTS), 1)).astype(jnp.float32)
        tri_lo = (col_i <= row_i).astype(jnp.float32)
        cum_c = jnp.dot(tri_lo, oh_c, preferred_element_type=jnp.float32)
        rank_c = jnp.sum(cum_c * oh_c, axis=1, keepdims=True
                         ).astype(jnp.int32) - 1
        gslot_c = jnp.where(rank_c < CAP, SLOT * idx_c + rank_c, -1)

        slot_iota = lax.broadcasted_iota(jnp.int32, (N_TOK, BLOCK), 1)

        def scatter_mat(q, dtype=jnp.float32):
            return (slot_iota + q * BLOCK == gslot_c).astype(dtype)

        pr = scatter_mat(r)
        out_ref[:, :] = jnp.dot(pr[:, 0:HALF], half0,
                                preferred_element_type=jnp.float32)
        out_ref[:, :] = out_ref[:, :] + jnp.dot(
            pr[:, HALF:BLOCK], half1, preferred_element_type=jnp.float32)

        for h in range(N_HOPS):
            q = lax.rem(r + 2 * N_DEV - 1 - h, N_DEV)
            pq = scatter_mat(q, jnp.bfloat16)
            for c in range(2):
                rdmas[(h, c)].wait_recv()
                if h + 1 < N_HOPS:
                    rdmas[(h + 1, c)] = hop(h + 1, c)
                out_ref[:, :] = out_ref[:, :] + jnp.dot(
                    pq[:, c * HALF:(c + 1) * HALF],
                    comm_ref[h + 1, c * HALF:(c + 1) * HALF, :],
                    preferred_element_type=jnp.float32,
                )

        for rdma in rdmas.values():
            rdma.wait_send()

    return pl.pallas_call(
        body,
        out_shape=jax.ShapeDtypeStruct((N_TOK, D_OUT), jnp.float32),
        in_specs=[
            pl.BlockSpec(memory_space=pltpu.VMEM),
            pl.BlockSpec(memory_space=pltpu.VMEM),
            pl.BlockSpec(memory_space=pltpu.VMEM),
            pl.BlockSpec(memory_space=pltpu.VMEM),
        ],
        out_specs=pl.BlockSpec(memory_space=pltpu.VMEM),
        scratch_shapes=[
            pltpu.VMEM((N_DEV, BLOCK, D_OUT), jnp.bfloat16),
            pltpu.SemaphoreType.DMA((N_HOPS, 2)),
            pltpu.SemaphoreType.DMA((N_HOPS, 2)),
        ],
        compiler_params=pltpu.CompilerParams(collective_id=0),
    )(x, expert_W, route_idx, route_idx_row)


# device time: 25044 ns/iter; 3.9697x vs baseline; 1.1661x over previous
import jax
import jax.numpy as jnp
from jax import lax
from jax.experimental import pallas as pl
from jax.experimental.pallas import tpu as pltpu

N_DEV = 4
N_EXPERTS = 16
E_LOCAL = N_EXPERTS // N_DEV
N_TOK = 1024
D_IN = 512
D_OUT = 1024
CAP = 51
SLOT = 56
BLOCK = E_LOCAL * SLOT
HALF = BLOCK // 2
N_HOPS = N_DEV - 1


def kernel(x, router_W, route_idx, expert_W):
    del router_W
    route_idx_row = route_idx.reshape(1, N_TOK)

    def body(x_ref, w_ref, sc_ref, sr_ref, out_ref, comm_ref,
             send_sems, recv_sems):
        r = lax.axis_index("i")
        left = lax.rem(r + N_DEV - 1, N_DEV)
        right = lax.rem(r + 1, N_DEV)

        barrier_sem = pltpu.get_barrier_semaphore()
        for nbr in (left, right, lax.rem(r + 2, N_DEV)):
            pl.semaphore_signal(
                barrier_sem, inc=1,
                device_id=(nbr,), device_id_type=pl.DeviceIdType.MESH,
            )

        idx_c = sc_ref[:, :]
        idx_r = sr_ref[:, :]
        row_i = lax.broadcasted_iota(jnp.int32, (N_TOK, N_TOK), 0)
        col_i = lax.broadcasted_iota(jnp.int32, (N_TOK, N_TOK), 1)

        oh_r = (idx_r == lax.broadcasted_iota(
            jnp.int32, (N_EXPERTS, N_TOK), 0)).astype(jnp.float32)
        tri_up = (row_i <= col_i).astype(jnp.float32)
        cum_r = jnp.dot(oh_r, tri_up, preferred_element_type=jnp.float32)
        rank_r = jnp.sum(cum_r * oh_r, axis=0, keepdims=True
                         ).astype(jnp.int32) - 1
        gslot_r = jnp.where(rank_r < CAP, SLOT * idx_r + rank_r, -1)

        slot_iota_r = lax.broadcasted_iota(jnp.int32, (BLOCK, N_TOK), 0)
        g = (slot_iota_r + r * BLOCK == gslot_r).astype(jnp.float32)

        cx = jnp.dot(g, x_ref[:, :],
                     preferred_element_type=jnp.float32)

        def expert_rows(j):
            return jnp.dot(cx[j * SLOT:(j + 1) * SLOT, :], w_ref[j],
                           preferred_element_type=jnp.float32)

        SENDS = ((1, 3), (3, 1), (2, 2))

        def send_to(delta, slot, c):
            rows = pl.ds(c * HALF, HALF)
            rdma = pltpu.make_async_remote_copy(
                src_ref=comm_ref.at[0, rows],
                dst_ref=comm_ref.at[slot, rows],
                send_sem=send_sems.at[slot - 1, c],
                recv_sem=recv_sems.at[slot - 1, c],
                device_id=(lax.rem(r + delta, N_DEV),),
                device_id_type=pl.DeviceIdType.MESH,
            )
            rdma.start()
            return rdma

        half0 = jnp.concatenate([expert_rows(0), expert_rows(1)], axis=0)
        comm_ref[0, 0:HALF, :] = half0.astype(jnp.bfloat16)
        pl.semaphore_wait(barrier_sem, 3)
        rdmas = {(slot, 0): send_to(d, slot, 0) for d, slot in SENDS}

        half1 = jnp.concatenate([expert_rows(2), expert_rows(3)], axis=0)
        comm_ref[0, HALF:BLOCK, :] = half1.astype(jnp.bfloat16)
        for d, slot in SENDS:
            rdmas[(slot, 1)] = send_to(d, slot, 1)

        oh_c = (idx_c == lax.broadcasted_iota(
            jnp.int32, (N_TOK, N_EXPERTS), 1)).astype(jnp.float32)
        tri_lo = (col_i <= row_i).astype(jnp.float32)
        cum_c = jnp.dot(tri_lo, oh_c, preferred_element_type=jnp.float32)
        rank_c = jnp.sum(cum_c * oh_c, axis=1, keepdims=True
                         ).astype(jnp.int32) - 1
        gslot_c = jnp.where(rank_c < CAP, SLOT * idx_c + rank_c, -1)

        slot_iota = lax.broadcasted_iota(jnp.int32, (N_TOK, BLOCK), 1)

        def scatter_mat(q, dtype=jnp.float32):
            return (slot_iota + q * BLOCK == gslot_c).astype(dtype)

        pr = scatter_mat(r)
        out_ref[:, :] = jnp.dot(pr[:, 0:HALF], half0,
                                preferred_element_type=jnp.float32)
        out_ref[:, :] = out_ref[:, :] + jnp.dot(
            pr[:, HALF:BLOCK], half1, preferred_element_type=jnp.float32)

        for slot in (3, 1, 2):
            rdmas[(slot, 0)].wait_recv()
            rdmas[(slot, 1)].wait_recv()
            q = lax.rem(r + slot, N_DEV)
            out_ref[:, :] = out_ref[:, :] + jnp.dot(
                scatter_mat(q, jnp.bfloat16), comm_ref[slot],
                preferred_element_type=jnp.float32,
            )

        for rdma in rdmas.values():
            rdma.wait_send()

    return pl.pallas_call(
        body,
        out_shape=jax.ShapeDtypeStruct((N_TOK, D_OUT), jnp.float32),
        in_specs=[
            pl.BlockSpec(memory_space=pltpu.VMEM),
            pl.BlockSpec(memory_space=pltpu.VMEM),
            pl.BlockSpec(memory_space=pltpu.VMEM),
            pl.BlockSpec(memory_space=pltpu.VMEM),
        ],
        out_specs=pl.BlockSpec(memory_space=pltpu.VMEM),
        scratch_shapes=[
            pltpu.VMEM((N_DEV, BLOCK, D_OUT), jnp.bfloat16),
            pltpu.SemaphoreType.DMA((N_HOPS, 2)),
            pltpu.SemaphoreType.DMA((N_HOPS, 2)),
        ],
        compiler_params=pltpu.CompilerParams(collective_id=0),
    )(x, expert_W, route_idx, route_idx_row)


# device time: 24904 ns/iter; 3.9920x vs baseline; 1.0056x over previous
import jax
import jax.numpy as jnp
from jax import lax
from jax.experimental import pallas as pl
from jax.experimental.pallas import tpu as pltpu

N_DEV = 4
N_EXPERTS = 16
E_LOCAL = N_EXPERTS // N_DEV
N_TOK = 1024
D_IN = 512
D_OUT = 1024
CAP = 51
SLOT = 56
BLOCK = E_LOCAL * SLOT
HALF = BLOCK // 2


def kernel(x, router_W, route_idx, expert_W):
    del router_W
    route_idx_row = route_idx.reshape(1, N_TOK)

    def body(x_hbm, w_hbm, sr_ref, out_ref, xv_ref, wv_ref, comm_ref,
             send_sems, recv_sems, load_sems):
        r = lax.axis_index("i")
        left = lax.rem(r + N_DEV - 1, N_DEV)
        right = lax.rem(r + 1, N_DEV)

        x_load = pltpu.make_async_copy(x_hbm, xv_ref, load_sems.at[0])
        w_load = pltpu.make_async_copy(w_hbm, wv_ref, load_sems.at[1])
        x_load.start()
        w_load.start()

        barrier_sem = pltpu.get_barrier_semaphore()
        for nbr in (left, right, lax.rem(r + 2, N_DEV)):
            pl.semaphore_signal(
                barrier_sem, inc=1,
                device_id=(nbr,), device_id_type=pl.DeviceIdType.MESH,
            )

        idx_r = sr_ref[:, :]
        row_i = lax.broadcasted_iota(jnp.int32, (N_TOK, N_TOK), 0)
        col_i = lax.broadcasted_iota(jnp.int32, (N_TOK, N_TOK), 1)

        oh_r = (idx_r == lax.broadcasted_iota(
            jnp.int32, (N_EXPERTS, N_TOK), 0)).astype(jnp.float32)
        tri_up = (row_i <= col_i).astype(jnp.float32)
        cum_r = jnp.dot(oh_r, tri_up, preferred_element_type=jnp.float32)
        rank_r = jnp.sum(cum_r * oh_r, axis=0, keepdims=True
                         ).astype(jnp.int32) - 1
        gslot_r = jnp.where(rank_r < CAP, SLOT * idx_r + rank_r, -1)

        slot_iota_r = lax.broadcasted_iota(jnp.int32, (BLOCK, N_TOK), 0)
        g = (slot_iota_r + r * BLOCK == gslot_r).astype(jnp.float32)

        x_load.wait()
        cx = jnp.dot(g, xv_ref[:, :],
                     preferred_element_type=jnp.float32)
        w_load.wait()

        def expert_rows(j):
            return jnp.dot(cx[j * SLOT:(j + 1) * SLOT, :], wv_ref[j],
                           preferred_element_type=jnp.float32)

        SENDS = ((1, 3), (3, 1), (2, 2))

        def send_to(delta, slot, c):
            rows = pl.ds(c * HALF, HALF)
            rdma = pltpu.make_async_remote_copy(
                src_ref=comm_ref.at[0, rows],
                dst_ref=comm_ref.at[slot, rows],
                send_sem=send_sems.at[slot - 1, c],
                recv_sem=recv_sems.at[slot - 1, c],
                device_id=(lax.rem(r + delta, N_DEV),),
                device_id_type=pl.DeviceIdType.MESH,
            )
            rdma.start()
            return rdma

        half0 = jnp.concatenate([expert_rows(0), expert_rows(1)], axis=0)
        comm_ref[0, 0:HALF, :] = half0.astype(jnp.bfloat16)
        pl.semaphore_wait(barrier_sem, 3)
        rdmas = {(slot, 0): send_to(d, slot, 0) for d, slot in SENDS}

        half1 = jnp.concatenate([expert_rows(2), expert_rows(3)], axis=0)
        comm_ref[0, HALF:BLOCK, :] = half1.astype(jnp.bfloat16)
        for d, slot in SENDS:
            rdmas[(slot, 1)] = send_to(d, slot, 1)

        gr_f = gslot_r.astype(jnp.float32)
        diag = jnp.where(row_i == col_i,
                         jnp.broadcast_to(gr_f, (N_TOK, N_TOK)), 0.0)
        gslot_c = jnp.sum(diag, axis=1, keepdims=True)

        slot_iota = lax.broadcasted_iota(
            jnp.int32, (N_TOK, BLOCK), 1).astype(jnp.float32)

        def scatter_mat(q, dtype=jnp.float32):
            qoff = (q * BLOCK).astype(jnp.float32)
            return (slot_iota + qoff == gslot_c).astype(dtype)

        pr = scatter_mat(r)
        out_ref[:, :] = jnp.dot(pr[:, 0:HALF], half0,
                                preferred_element_type=jnp.float32)
        out_ref[:, :] = out_ref[:, :] + jnp.dot(
            pr[:, HALF:BLOCK], half1, preferred_element_type=jnp.float32)

        for slot in (3, 1, 2):
            rdmas[(slot, 0)].wait_recv()
            rdmas[(slot, 1)].wait_recv()
            q = lax.rem(r + slot, N_DEV)
            out_ref[:, :] = out_ref[:, :] + jnp.dot(
                scatter_mat(q, jnp.bfloat16), comm_ref[slot],
                preferred_element_type=jnp.float32,
            )

        for rdma in rdmas.values():
            rdma.wait_send()

    return pl.pallas_call(
        body,
        out_shape=jax.ShapeDtypeStruct((N_TOK, D_OUT), jnp.float32),
        in_specs=[
            pl.BlockSpec(memory_space=pl.ANY),
            pl.BlockSpec(memory_space=pl.ANY),
            pl.BlockSpec(memory_space=pltpu.VMEM),
        ],
        out_specs=pl.BlockSpec(memory_space=pltpu.VMEM),
        scratch_shapes=[
            pltpu.VMEM((N_TOK, D_IN), jnp.float32),
            pltpu.VMEM((E_LOCAL, D_IN, D_OUT), jnp.float32),
            pltpu.VMEM((N_DEV, BLOCK, D_OUT), jnp.bfloat16),
            pltpu.SemaphoreType.DMA((N_DEV - 1, 2)),
            pltpu.SemaphoreType.DMA((N_DEV - 1, 2)),
            pltpu.SemaphoreType.DMA((2,)),
        ],
        compiler_params=pltpu.CompilerParams(collective_id=0),
    )(x, expert_W, route_idx_row)


# device time: 22412 ns/iter; 4.4359x vs baseline; 1.1112x over previous
import jax
import jax.numpy as jnp
from jax import lax
from jax.experimental import pallas as pl
from jax.experimental.pallas import tpu as pltpu

N_DEV = 4
N_EXPERTS = 16
E_LOCAL = N_EXPERTS // N_DEV
N_TOK = 1024
D_IN = 512
D_OUT = 1024
CAP = 51
SLOT = 56
BLOCK = E_LOCAL * SLOT
HALF = BLOCK // 2


def kernel(x, router_W, route_idx, expert_W):
    del router_W
    route_idx_row = route_idx.reshape(1, N_TOK)

    def body(x_hbm, w_hbm, sr_ref, out_ref, xv_ref, wv_ref, comm_ref,
             send_sems, recv_sems, load_sems):
        r = lax.axis_index("i")
        left = lax.rem(r + N_DEV - 1, N_DEV)
        right = lax.rem(r + 1, N_DEV)

        x_load = pltpu.make_async_copy(x_hbm, xv_ref, load_sems.at[0])
        w_load = pltpu.make_async_copy(w_hbm, wv_ref, load_sems.at[1])
        x_load.start()
        w_load.start()

        barrier_sem = pltpu.get_barrier_semaphore()
        for nbr in (left, right, lax.rem(r + 2, N_DEV)):
            pl.semaphore_signal(
                barrier_sem, inc=1,
                device_id=(nbr,), device_id_type=pl.DeviceIdType.MESH,
            )

        idx_r = sr_ref[:, :]
        row_i = lax.broadcasted_iota(jnp.int32, (N_TOK, N_TOK), 0)
        col_i = lax.broadcasted_iota(jnp.int32, (N_TOK, N_TOK), 1)

        oh_r = (idx_r == lax.broadcasted_iota(
            jnp.int32, (N_EXPERTS, N_TOK), 0)).astype(jnp.float32)
        tri_up = (row_i <= col_i).astype(jnp.float32)
        cum_r = jnp.dot(oh_r, tri_up, preferred_element_type=jnp.float32)
        rank_r = jnp.sum(cum_r * oh_r, axis=0, keepdims=True
                         ).astype(jnp.int32) - 1
        gslot_r = jnp.where(rank_r < CAP, SLOT * idx_r + rank_r, -1)

        slot_iota_r = lax.broadcasted_iota(jnp.int32, (BLOCK, N_TOK), 0)
        g = (slot_iota_r + r * BLOCK == gslot_r).astype(jnp.float32)

        x_load.wait()
        cx = jnp.dot(g, xv_ref[:, :],
                     preferred_element_type=jnp.float32)
        w_load.wait()

        def expert_rows(j):
            return jnp.dot(cx[j * SLOT:(j + 1) * SLOT, :], wv_ref[j],
                           preferred_element_type=jnp.float32)

        SENDS = ((1, 3), (3, 1), (2, 2))

        def send_to(delta, slot, c):
            rows = pl.ds(c * HALF, HALF)
            rdma = pltpu.make_async_remote_copy(
                src_ref=comm_ref.at[0, rows],
                dst_ref=comm_ref.at[slot, rows],
                send_sem=send_sems.at[slot - 1, c],
                recv_sem=recv_sems.at[slot - 1, c],
                device_id=(lax.rem(r + delta, N_DEV),),
                device_id_type=pl.DeviceIdType.MESH,
            )
            rdma.start()
            return rdma

        half0 = jnp.concatenate([expert_rows(0), expert_rows(1)], axis=0)
        comm_ref[0, 0:HALF, :] = half0.astype(jnp.bfloat16)
        pl.semaphore_wait(barrier_sem, 3)
        rdmas = {(slot, 0): send_to(d, slot, 0) for d, slot in SENDS}

        half1 = jnp.concatenate([expert_rows(2), expert_rows(3)], axis=0)
        comm_ref[0, HALF:BLOCK, :] = half1.astype(jnp.bfloat16)
        for d, slot in SENDS:
            rdmas[(slot, 1)] = send_to(d, slot, 1)

        gr_f = gslot_r.astype(jnp.float32)
        diag = jnp.where(row_i == col_i,
                         jnp.broadcast_to(gr_f, (N_TOK, N_TOK)), 0.0)
        gslot_c = jnp.sum(diag, axis=1, keepdims=True)

        slot_iota = lax.broadcasted_iota(
            jnp.int32, (N_TOK, BLOCK), 1).astype(jnp.float32)

        def scatter_mat(q, dtype=jnp.float32):
            qoff = (q * BLOCK).astype(jnp.float32)
            return (slot_iota + qoff == gslot_c).astype(dtype)

        pr = scatter_mat(r)
        out_ref[:, :] = jnp.dot(pr[:, 0:HALF], half0,
                                preferred_element_type=jnp.float32)
        out_ref[:, :] = out_ref[:, :] + jnp.dot(
            pr[:, HALF:BLOCK], half1, preferred_element_type=jnp.float32)

        for slot in (3, 1, 2):
            rdmas[(slot, 0)].wait_recv()
            rdmas[(slot, 1)].wait_recv()
            q = lax.rem(r + slot, N_DEV)
            out_ref[:, :] = out_ref[:, :] + jnp.dot(
                scatter_mat(q, jnp.bfloat16), comm_ref[slot],
                preferred_element_type=jnp.float32,
            )

        for rdma in rdmas.values():
            rdma.wait_send()

    return pl.pallas_call(
        body,
        out_shape=jax.ShapeDtypeStruct((N_TOK, D_OUT), jnp.float32),
        in_specs=[
            pl.BlockSpec(memory_space=pl.ANY),
            pl.BlockSpec(memory_space=pl.ANY),
            pl.BlockSpec(memory_space=pltpu.VMEM),
        ],
        out_specs=pl.BlockSpec(memory_space=pltpu.VMEM),
        scratch_shapes=[
            pltpu.VMEM((N_TOK, D_IN), jnp.float32),
            pltpu.VMEM((E_LOCAL, D_IN, D_OUT), jnp.float32),
            pltpu.VMEM((N_DEV, BLOCK, D_OUT), jnp.bfloat16),
            pltpu.SemaphoreType.DMA((N_DEV - 1, 2)),
            pltpu.SemaphoreType.DMA((N_DEV - 1, 2)),
            pltpu.SemaphoreType.DMA((2,)),
        ],
        compiler_params=pltpu.CompilerParams(collective_id=0),
    )(
        pltpu.with_memory_space_constraint(x, pltpu.MemorySpace.HBM),
        pltpu.with_memory_space_constraint(expert_W, pltpu.MemorySpace.HBM),
        route_idx_row,
    )


# device time: 21164 ns/iter; 4.6975x vs baseline; 1.0590x over previous
import jax
import jax.numpy as jnp
from jax import lax
from jax.experimental import pallas as pl
from jax.experimental.pallas import tpu as pltpu

N_DEV = 4
N_EXPERTS = 16
E_LOCAL = N_EXPERTS // N_DEV
N_TOK = 1024
D_IN = 512
D_OUT = 1024
CAP = 51
SLOT = 56
BLOCK = E_LOCAL * SLOT
HALF = BLOCK // 2


def kernel(x, router_W, route_idx, expert_W):
    del router_W
    route_idx_row = route_idx.reshape(1, N_TOK).astype(jnp.float32)

    def body(x_hbm, w_hbm, sr_ref, out_ref, xv_ref, wv_ref, comm_ref,
             send_sems, recv_sems, load_sems):
        r = lax.axis_index("i")
        left = lax.rem(r + N_DEV - 1, N_DEV)
        right = lax.rem(r + 1, N_DEV)

        x_load = pltpu.make_async_copy(x_hbm, xv_ref, load_sems.at[0])
        x_load.start()
        w_loads = []
        for j in range(E_LOCAL):
            w_loads.append(pltpu.make_async_copy(
                w_hbm.at[j], wv_ref.at[j], load_sems.at[1 + j]))
            w_loads[j].start()

        barrier_sem = pltpu.get_barrier_semaphore()
        for nbr in (left, right, lax.rem(r + 2, N_DEV)):
            pl.semaphore_signal(
                barrier_sem, inc=1,
                device_id=(nbr,), device_id_type=pl.DeviceIdType.MESH,
            )

        idx_r = sr_ref[:, :]
        row_i = lax.broadcasted_iota(jnp.int32, (N_TOK, N_TOK), 0)
        col_i = lax.broadcasted_iota(jnp.int32, (N_TOK, N_TOK), 1)

        oh_r = (idx_r == lax.broadcasted_iota(
            jnp.int32, (N_EXPERTS, N_TOK), 0).astype(jnp.float32)
                ).astype(jnp.float32)
        tri_up = (row_i <= col_i).astype(jnp.float32)
        cum_r = jnp.dot(oh_r, tri_up, preferred_element_type=jnp.float32)
        rank_r = jnp.sum(cum_r * oh_r, axis=0, keepdims=True) - 1.0
        gslot_r = jnp.where(rank_r < CAP, SLOT * idx_r + rank_r, -1.0)

        slot_iota_r = lax.broadcasted_iota(
            jnp.int32, (BLOCK, N_TOK), 0).astype(jnp.float32)
        rB = (r * BLOCK).astype(jnp.float32)
        g = (slot_iota_r + rB == gslot_r).astype(jnp.float32)

        x_load.wait()
        cx = jnp.dot(g, xv_ref[:, :],
                     preferred_element_type=jnp.float32)

        def expert_rows(j):
            w_loads[j].wait()
            return jnp.dot(cx[j * SLOT:(j + 1) * SLOT, :], wv_ref[j],
                           preferred_element_type=jnp.float32)

        SENDS = ((1, 3), (3, 1), (2, 2))

        def send_to(delta, slot, c):
            rows = pl.ds(c * HALF, HALF)
            rdma = pltpu.make_async_remote_copy(
                src_ref=comm_ref.at[0, rows],
                dst_ref=comm_ref.at[slot, rows],
                send_sem=send_sems.at[slot - 1, c],
                recv_sem=recv_sems.at[slot - 1, c],
                device_id=(lax.rem(r + delta, N_DEV),),
                device_id_type=pl.DeviceIdType.MESH,
            )
            rdma.start()
            return rdma

        half0 = jnp.concatenate([expert_rows(0), expert_rows(1)], axis=0)
        comm_ref[0, 0:HALF, :] = half0.astype(jnp.bfloat16)
        pl.semaphore_wait(barrier_sem, 3)
        rdmas = {(slot, 0): send_to(d, slot, 0) for d, slot in SENDS}

        half1 = jnp.concatenate([expert_rows(2), expert_rows(3)], axis=0)
        comm_ref[0, HALF:BLOCK, :] = half1.astype(jnp.bfloat16)
        for d, slot in SENDS:
            rdmas[(slot, 1)] = send_to(d, slot, 1)

        diag = jnp.where(row_i == col_i,
                         jnp.broadcast_to(gslot_r, (N_TOK, N_TOK)), 0.0)
        gslot_c = jnp.sum(diag, axis=1, keepdims=True)

        slot_iota = lax.broadcasted_iota(
            jnp.int32, (N_TOK, BLOCK), 1).astype(jnp.float32)

        def scatter_mat(q, dtype=jnp.float32):
            qoff = (q * BLOCK).astype(jnp.float32)
            return (slot_iota + qoff == gslot_c).astype(dtype)

        pr = scatter_mat(r)
        out_ref[:, :] = jnp.dot(pr[:, 0:HALF], half0,
                                preferred_element_type=jnp.float32)
        out_ref[:, :] = out_ref[:, :] + jnp.dot(
            pr[:, HALF:BLOCK], half1, preferred_element_type=jnp.float32)

        for slot in (3, 1):
            rdmas[(slot, 0)].wait_recv()
            rdmas[(slot, 1)].wait_recv()
            q = lax.rem(r + slot, N_DEV)
            out_ref[:, :] = out_ref[:, :] + jnp.dot(
                scatter_mat(q, jnp.bfloat16), comm_ref[slot],
                preferred_element_type=jnp.float32,
            )
        q2 = lax.rem(r + 2, N_DEV)
        p2 = scatter_mat(q2, jnp.bfloat16)
        for c in range(2):
            rdmas[(2, c)].wait_recv()
            out_ref[:, :] = out_ref[:, :] + jnp.dot(
                p2[:, c * HALF:(c + 1) * HALF],
                comm_ref[2, c * HALF:(c + 1) * HALF, :],
                preferred_element_type=jnp.float32,
            )

        for rdma in rdmas.values():
            rdma.wait_send()

    return pl.pallas_call(
        body,
        out_shape=jax.ShapeDtypeStruct((N_TOK, D_OUT), jnp.float32),
        in_specs=[
            pl.BlockSpec(memory_space=pl.ANY),
            pl.BlockSpec(memory_space=pl.ANY),
            pl.BlockSpec(memory_space=pltpu.VMEM),
        ],
        out_specs=pl.BlockSpec(memory_space=pltpu.VMEM),
        scratch_shapes=[
            pltpu.VMEM((N_TOK, D_IN), jnp.float32),
            pltpu.VMEM((E_LOCAL, D_IN, D_OUT), jnp.float32),
            pltpu.VMEM((N_DEV, BLOCK, D_OUT), jnp.bfloat16),
            pltpu.SemaphoreType.DMA((N_DEV - 1, 2)),
            pltpu.SemaphoreType.DMA((N_DEV - 1, 2)),
            pltpu.SemaphoreType.DMA((1 + E_LOCAL,)),
        ],
        compiler_params=pltpu.CompilerParams(collective_id=0),
    )(
        pltpu.with_memory_space_constraint(x, pltpu.MemorySpace.HBM),
        pltpu.with_memory_space_constraint(expert_W, pltpu.MemorySpace.HBM),
        route_idx_row,
    )
